# baseline (speedup 1.0000x reference)
import functools

import jax
import jax.numpy as jnp
import numpy as np
from jax import lax
from jax.experimental import pallas as pl
from jax.experimental.pallas import tpu as pltpu
from jax.experimental.pallas import tpu_sc as plsc

N = 2
LQ = 4096
DM = 256
NHEAD = 8
DH = DM // NHEAD
NLVL = 4
NPTS = 4
SHAPES = (4096, 2048, 1024, 512)
STARTS = (0, 4096, 6144, 7168)
LEN_IN = 7680

NQ = N * LQ
NQH = NQ * NHEAD
NROWS_V = N * LEN_IN * NHEAD

_NC, _NS = 2, 16
_NW = _NC * _NS
_ROWS_PER_W = NQH // _NW
_CB = 32
_NCHUNK = _ROWS_PER_W // _CB
_SPC = _CB * NLVL * NPTS
_QPC = _CB // NHEAD
_EPC = _QPC * 2 * 128
_GROWS = _SPC // 128

_lane = np.arange(128)
_lvl = (_lane // 4) % 4
_TVEC = np.array(SHAPES, np.float32)[_lvl]
_STARTH = (np.array(STARTS, np.int64)[_lvl] * NHEAD).astype(np.int32)
_HLANE = (_lane // 16).astype(np.int32)
_BD = (_lane[:, None] // 16 == _lane[None, :] // 16).astype(np.float32)

_t = np.arange(LEN_IN)
_ends = np.array([4096, 6144, 7168, 7680])
_lvl_of_t = np.searchsorted(_ends, _t, side="right")
_TNEXT = np.minimum(_t + 1, _ends[_lvl_of_t] - 1).astype(np.int32)


def _mm_bias_body(x_ref, w_ref, b_ref, o_ref):
    o_ref[...] = (
        jnp.dot(x_ref[...], w_ref[...], preferred_element_type=jnp.float32,
                precision=jax.lax.Precision.HIGHEST)
        + b_ref[...]
    ).astype(o_ref.dtype)


def _mm_bias(x, w_t, b, bm, out_dtype=jnp.float32):
    m = x.shape[0]
    k = x.shape[1]
    n_out = w_t.shape[1]
    return pl.pallas_call(
        _mm_bias_body,
        grid=(m // bm,),
        in_specs=[
            pl.BlockSpec((bm, k), lambda i: (i, 0)),
            pl.BlockSpec((k, n_out), lambda i: (0, 0)),
            pl.BlockSpec((1, n_out), lambda i: (0, 0)),
        ],
        out_specs=pl.BlockSpec((bm, n_out), lambda i: (i, 0)),
        out_shape=jax.ShapeDtypeStruct((m, n_out), out_dtype),
    )(x, w_t, b.reshape(1, n_out))


_PREP_BM = 1024


def _prep_body(q_ref, rp_ref, wst_ref, bs_ref, wat_ref, ba_ref,
               tv_ref, sh_ref, hl_ref, bd_ref,
               c01_ref, i0_ref):
    pid = pl.program_id(0)
    q = q_ref[...]
    off = jnp.dot(q, wst_ref[...], preferred_element_type=jnp.float32,
                  precision=jax.lax.Precision.HIGHEST) + bs_ref[...]
    logits = jnp.dot(q, wat_ref[...], preferred_element_type=jnp.float32,
                     precision=jax.lax.Precision.HIGHEST) + ba_ref[...]
    m = jnp.max(logits, axis=-1, keepdims=True)
    e = jnp.exp(logits - m)
    denom = jnp.dot(e, bd_ref[...], preferred_element_type=jnp.float32,
                    precision=jax.lax.Precision.HIGHEST)
    aw = e / denom
    refb = rp_ref[...]
    tvec = tv_ref[...]
    loc = refb + off / tvec
    ix = jnp.clip(loc * tvec - 0.5, 0.0, tvec - 1.0)
    i0f = jnp.floor(ix)
    w = ix - i0f
    i0 = i0f.astype(jnp.int32)
    nbase = (pid // (LQ // _PREP_BM)) * (LEN_IN * NHEAD)
    idx0 = nbase + sh_ref[...] + i0 * NHEAD + hl_ref[...]
    c01_ref[...] = jnp.concatenate([aw * (1.0 - w), aw * w], axis=1)
    i0_ref[...] = idx0


def _prep(q2, rp128, ws_t, bs, wa_t, ba):
    vec_spec = pl.BlockSpec((1, 128), lambda i: (0, 0))
    blk128 = pl.BlockSpec((_PREP_BM, 128), lambda i: (i, 0))
    blk256 = pl.BlockSpec((_PREP_BM, 256), lambda i: (i, 0))
    return pl.pallas_call(
        _prep_body,
        grid=(NQ // _PREP_BM,),
        in_specs=[
            pl.BlockSpec((_PREP_BM, DM), lambda i: (i, 0)),
            blk128,
            pl.BlockSpec((DM, 128), lambda i: (0, 0)),
            vec_spec,
            pl.BlockSpec((DM, 128), lambda i: (0, 0)),
            vec_spec,
            vec_spec, vec_spec, vec_spec,
            pl.BlockSpec((128, 128), lambda i: (0, 0)),
        ],
        out_specs=[blk256, blk128],
        out_shape=[
            jax.ShapeDtypeStruct((NQ, 256), jnp.float32),
            jax.ShapeDtypeStruct((NQ, 128), jnp.int32),
        ],
    )(q2, rp128, ws_t, bs.reshape(1, 128), wa_t, ba.reshape(1, 128),
      _TVEC.reshape(1, 128), _STARTH.reshape(1, 128), _HLANE.reshape(1, 128),
      _BD)


_CPS = 8
_NSUP = _NCHUNK // _CPS
_SQROWS = _QPC * _CPS
_SIROWS = _SQROWS
_SELEM = _SQROWS * 256


@functools.cache
def _get_sc_sample():
    mesh = plsc.VectorSubcoreMesh(core_axis_name="c", subcore_axis_name="s")
    stage = lambda: (
        pltpu.VMEM((_SIROWS, 128), jnp.int32),
        pltpu.VMEM((_SELEM,), jnp.float32),
        pltpu.SemaphoreType.DMA,
    )

    @functools.partial(
        pl.kernel,
        mesh=mesh,
        compiler_params=pltpu.CompilerParams(
            needs_layout_passes=False, use_tc_tiling_on_sc=False),
        out_type=jax.ShapeDtypeStruct((NQH, DH), jnp.float32),
        scratch_types=[
            *stage(), *stage(),
            pltpu.VMEM((_SPC, 2 * DH), jnp.float32), pltpu.SemaphoreType.DMA,
            pltpu.VMEM((_SPC, 2 * DH), jnp.float32), pltpu.SemaphoreType.DMA,
            pltpu.VMEM((_CB, DH), jnp.float32), pltpu.SemaphoreType.DMA,
            pltpu.VMEM((_CB, DH), jnp.float32), pltpu.SemaphoreType.DMA,
        ],
    )
    def _sc_sample(v_hbm, i01_hbm, c01_hbm, out_hbm,
                   iA, cA, sA, iB, cB, sB,
                   r0, rs0, r1, rs1, o0, os0, o1, os1):
        _sc_body(v_hbm, i01_hbm, c01_hbm, out_hbm,
                 ((iA, cA, sA), (iB, cB, sB)),
                 ((r0, rs0), (r1, rs1)), ((o0, os0), (o1, os1)))

    return _sc_sample


def _sc_body(v_hbm, i01_hbm, c01_hbm, out_hbm, stages, rows, outs):
    wid = lax.axis_index("s") * _NC + lax.axis_index("c")
    base_row_w = wid * _ROWS_PER_W
    qrow_w = wid * (_ROWS_PER_W // NHEAD)

    def stage_slices(s):
        qrow0 = pl.multiple_of(qrow_w + s * _SQROWS, _SQROWS)
        return (i01_hbm.at[pl.ds(qrow0, _SIROWS)],
                c01_hbm.at[pl.ds(qrow0 * 256, _SELEM)])

    def stage_start(s, sbuf):
        i01_v, c01_v, sem = sbuf
        isl, csl = stage_slices(s)
        pltpu.async_copy(isl, i01_v, sem)
        pltpu.async_copy(csl, c01_v, sem)

    def stage_wait(s, sbuf):
        i01_v, c01_v, sem = sbuf
        isl, csl = stage_slices(s)
        pltpu.make_async_copy(isl, i01_v, sem).wait()
        pltpu.make_async_copy(csl, c01_v, sem).wait()

    def fire(lc, sbuf, rbuf):
        i01_v, _, _ = sbuf
        rows_v, rsem = rbuf
        for j in range(_GROWS):
            pltpu.async_copy(v_hbm.at[i01_v.at[lc * _GROWS + j]],
                             rows_v.at[pl.ds(j * 128, 128)], rsem)

    def drain_blend(c, lc, sbuf, rbuf, obuf, first_store):
        i01_v, c01_v, _ = sbuf
        rows_v, rsem = rbuf
        out_v, osem = obuf
        base_row = pl.multiple_of(base_row_w + c * _CB, _CB)
        for j in range(_GROWS):
            pltpu.make_async_copy(v_hbm.at[i01_v.at[lc * _GROWS + j]],
                                  rows_v.at[pl.ds(j * 128, 128)], rsem).wait()

        @pl.when(jnp.logical_not(first_store))
        def _():
            pltpu.make_async_copy(out_v, out_hbm.at[pl.ds(base_row, _CB)],
                                  osem).wait()

        cbase = lc * _EPC

        def blend(r, _):
            e0 = cbase + (r // NHEAD) * 256 + (r % NHEAD) * 16
            rbase = (r // NHEAD) * 128 + (r % NHEAD) * 16
            zeros = jnp.zeros((16,), jnp.float32)
            acc_lo = [zeros, zeros, zeros, zeros]
            acc_hi = [zeros, zeros, zeros, zeros]
            for k in range(NLVL * NPTS):
                a = k % 4
                s0 = e0 + k
                s1 = s0 + 128
                c0vec = plsc.load_gather(c01_v, [jnp.full((16,), 0, jnp.int32) + s0])
                c1vec = plsc.load_gather(c01_v, [jnp.full((16,), 0, jnp.int32) + s1])
                r0 = rbase + k
                acc_lo[a] = (acc_lo[a] + c0vec * rows_v[r0, pl.ds(0, 16)]
                             + c1vec * rows_v[r0, pl.ds(32, 16)])
                acc_hi[a] = (acc_hi[a] + c0vec * rows_v[r0, pl.ds(16, 16)]
                             + c1vec * rows_v[r0, pl.ds(48, 16)])
            out_v[r, pl.ds(0, 16)] = (acc_lo[0] + acc_lo[1]) + (acc_lo[2] + acc_lo[3])
            out_v[r, pl.ds(16, 16)] = (acc_hi[0] + acc_hi[1]) + (acc_hi[2] + acc_hi[3])
            return 0

        lax.fori_loop(0, _CB, blend, 0, unroll=2)
        pltpu.async_copy(out_v, out_hbm.at[pl.ds(base_row, _CB)], osem)

    stage_start(0, stages[0])
    stage_start(1, stages[1])
    stage_wait(0, stages[0])
    fire(0, stages[0], rows[0])

    def super_block(sp, sup_par):
        s = 2 * sp + sup_par
        s_cur = stages[sup_par]
        s_next = stages[1 - sup_par]

        def g_step(g2, _):
            for t in (0, 1):
                lc = 2 * g2 + t
                c = s * _CPS + lc
                if t == 0:
                    fire(lc + 1, s_cur, rows[1])
                else:
                    @pl.when(g2 < 3)
                    def _():
                        fire(lc + 1, s_cur, rows[0])

                    @pl.when(jnp.logical_and(g2 == 3, s < _NSUP - 1))
                    def _():
                        stage_wait(s + 1, s_next)
                        fire(0, s_next, rows[0])
                drain_blend(c, lc, s_cur, rows[t], outs[t],
                            first_store=(c == t))
            return 0

        lax.fori_loop(0, _CPS // 2, g_step, 0)
        @pl.when(jnp.asarray(s + 2 < _NSUP))
        def _():
            stage_start(s + 2, s_cur)

    def sp_step(sp, _):
        super_block(sp, 0)
        super_block(sp, 1)
        return 0

    lax.fori_loop(0, _NSUP // 2, sp_step, 0)
    for t in (0, 1):
        out_v, osem = outs[t]
        base_row = pl.multiple_of(base_row_w, _CB)
        pltpu.make_async_copy(out_v, out_hbm.at[pl.ds(base_row, _CB)],
                              osem).wait()


def kernel(query, reference_points, input_flatten, input_spatial_shapes,
           input_level_start_index, Wv, bv, Ws, bs, Wa, ba, Wo, bo):
    x = input_flatten.reshape(N * LEN_IN, DM)
    v = _mm_bias(x, Wv.T, bv, 1024)
    v4 = v.reshape(N, LEN_IN, NHEAD, DH)
    w_tab = jnp.concatenate([v4, v4[:, jnp.asarray(_TNEXT)]], axis=-1)
    w_tab = w_tab.reshape(NROWS_V, 2 * DH)

    q2 = query.reshape(NQ, DM)
    rp2 = reference_points.reshape(NQ, NLVL)
    rp128 = jnp.tile(jnp.repeat(rp2, NPTS, axis=1), (1, NHEAD))
    c01, i0 = _prep(q2, rp128, Ws.T, bs, Wa.T, ba)

    s = _get_sc_sample()(w_tab, i0, c01.reshape(-1))

    out = _mm_bias(s.reshape(NQ, DM), Wo.T, bo, 1024)
    return out.reshape(N, LQ, DM)


# speedup vs baseline: 1.8052x; 1.8052x over previous
import functools

import jax
import jax.numpy as jnp
import numpy as np
from jax import lax
from jax.experimental import pallas as pl
from jax.experimental.pallas import tpu as pltpu
from jax.experimental.pallas import tpu_sc as plsc

N = 2
LQ = 4096
DM = 256
NHEAD = 8
DH = DM // NHEAD
NLVL = 4
NPTS = 4
SHAPES = (4096, 2048, 1024, 512)
STARTS = (0, 4096, 6144, 7168)
LEN_IN = 7680

NQ = N * LQ
NQH = NQ * NHEAD
NROWS_V = N * LEN_IN * NHEAD

_NC, _NS = 2, 16
_NW = _NC * _NS
_ROWS_PER_W = NQH // _NW
_CB = 32
_NCHUNK = _ROWS_PER_W // _CB
_SPC = _CB * NLVL * NPTS
_QPC = _CB // NHEAD
_EPC = _QPC * 2 * 128
_GROWS = _EPC // 128

_lane = np.arange(128)
_lvl = (_lane // 4) % 4
_TVEC = np.array(SHAPES, np.float32)[_lvl]
_STARTH = (np.array(STARTS, np.int64)[_lvl] * NHEAD).astype(np.int32)
_HLANE = (_lane // 16).astype(np.int32)
_BD = (_lane[:, None] // 16 == _lane[None, :] // 16).astype(np.float32)


def _mm_bias_body(x_ref, w_ref, b_ref, o_ref):
    o_ref[...] = (
        jnp.dot(x_ref[...], w_ref[...], preferred_element_type=jnp.float32,
                precision=jax.lax.Precision.HIGHEST)
        + b_ref[...]
    ).astype(o_ref.dtype)


def _mm_bias(x, w_t, b, bm, out_dtype=jnp.float32):
    m = x.shape[0]
    k = x.shape[1]
    n_out = w_t.shape[1]
    return pl.pallas_call(
        _mm_bias_body,
        grid=(m // bm,),
        in_specs=[
            pl.BlockSpec((bm, k), lambda i: (i, 0)),
            pl.BlockSpec((k, n_out), lambda i: (0, 0)),
            pl.BlockSpec((1, n_out), lambda i: (0, 0)),
        ],
        out_specs=pl.BlockSpec((bm, n_out), lambda i: (i, 0)),
        out_shape=jax.ShapeDtypeStruct((m, n_out), out_dtype),
    )(x, w_t, b.reshape(1, n_out))


_PREP_BM = 1024


def _prep_body(q_ref, rp_ref, wst_ref, bs_ref, wat_ref, ba_ref,
               tv_ref, sh_ref, hl_ref, bd_ref,
               c01_ref, i01_ref):
    pid = pl.program_id(0)
    q = q_ref[...]
    off = jnp.dot(q, wst_ref[...], preferred_element_type=jnp.float32,
                  precision=jax.lax.Precision.HIGHEST) + bs_ref[...]
    logits = jnp.dot(q, wat_ref[...], preferred_element_type=jnp.float32,
                     precision=jax.lax.Precision.HIGHEST) + ba_ref[...]
    m = jnp.max(logits, axis=-1, keepdims=True)
    e = jnp.exp(logits - m)
    denom = jnp.dot(e, bd_ref[...], preferred_element_type=jnp.float32,
                    precision=jax.lax.Precision.HIGHEST)
    aw = e / denom
    refb = rp_ref[...]
    tvec = tv_ref[...]
    loc = refb + off / tvec
    ix = jnp.clip(loc * tvec - 0.5, 0.0, tvec - 1.0)
    i0f = jnp.floor(ix)
    w = ix - i0f
    i0 = i0f.astype(jnp.int32)
    i1 = jnp.minimum(i0 + 1, tvec.astype(jnp.int32) - 1)
    nbase = (pid // (LQ // _PREP_BM)) * (LEN_IN * NHEAD)
    idx0 = nbase + sh_ref[...] + i0 * NHEAD + hl_ref[...]
    idx1 = nbase + sh_ref[...] + i1 * NHEAD + hl_ref[...]
    c01_ref[...] = jnp.concatenate([aw * (1.0 - w), aw * w], axis=1)
    i01_ref[...] = jnp.concatenate([idx0, idx1], axis=1)


def _prep(q2, rp128, ws_t, bs, wa_t, ba):
    vec_spec = pl.BlockSpec((1, 128), lambda i: (0, 0))
    blk128 = pl.BlockSpec((_PREP_BM, 128), lambda i: (i, 0))
    blk256 = pl.BlockSpec((_PREP_BM, 256), lambda i: (i, 0))
    return pl.pallas_call(
        _prep_body,
        grid=(NQ // _PREP_BM,),
        in_specs=[
            pl.BlockSpec((_PREP_BM, DM), lambda i: (i, 0)),
            blk128,
            pl.BlockSpec((DM, 128), lambda i: (0, 0)),
            vec_spec,
            pl.BlockSpec((DM, 128), lambda i: (0, 0)),
            vec_spec,
            vec_spec, vec_spec, vec_spec,
            pl.BlockSpec((128, 128), lambda i: (0, 0)),
        ],
        out_specs=[blk256, blk256],
        out_shape=[
            jax.ShapeDtypeStruct((NQ, 256), jnp.float32),
            jax.ShapeDtypeStruct((NQ, 256), jnp.int32),
        ],
    )(q2, rp128, ws_t, bs.reshape(1, 128), wa_t, ba.reshape(1, 128),
      _TVEC.reshape(1, 128), _STARTH.reshape(1, 128), _HLANE.reshape(1, 128),
      _BD)


_CPS = 8
_NSUP = _NCHUNK // _CPS
_SQROWS = _QPC * _CPS
_SIROWS = _SQROWS * 2
_SELEM = _SQROWS * 256


@functools.cache
def _get_sc_sample():
    mesh = plsc.VectorSubcoreMesh(core_axis_name="c", subcore_axis_name="s")
    stage = lambda: (
        pltpu.VMEM((_SIROWS, 128), jnp.int32),
        pltpu.VMEM((_SELEM,), jnp.float32),
        pltpu.SemaphoreType.DMA,
    )

    @functools.partial(
        pl.kernel,
        mesh=mesh,
        compiler_params=pltpu.CompilerParams(
            needs_layout_passes=False, use_tc_tiling_on_sc=False),
        out_type=jax.ShapeDtypeStruct((NQH, DH), jnp.float32),
        scratch_types=[
            *stage(), *stage(),
            pltpu.VMEM((_EPC, DH), jnp.float32), pltpu.SemaphoreType.DMA,
            pltpu.VMEM((_EPC, DH), jnp.float32), pltpu.SemaphoreType.DMA,
            pltpu.VMEM((_CB, DH), jnp.float32), pltpu.SemaphoreType.DMA,
            pltpu.VMEM((_CB, DH), jnp.float32), pltpu.SemaphoreType.DMA,
        ],
    )
    def _sc_sample(v_hbm, i01_hbm, c01_hbm, out_hbm,
                   iA, cA, sA, iB, cB, sB,
                   r0, rs0, r1, rs1, o0, os0, o1, os1):
        _sc_body(v_hbm, i01_hbm, c01_hbm, out_hbm,
                 ((iA, cA, sA), (iB, cB, sB)),
                 ((r0, rs0), (r1, rs1)), ((o0, os0), (o1, os1)))

    return _sc_sample


def _sc_body(v_hbm, i01_hbm, c01_hbm, out_hbm, stages, rows, outs):
    wid = lax.axis_index("s") * _NC + lax.axis_index("c")
    base_row_w = wid * _ROWS_PER_W
    qrow_w = wid * (_ROWS_PER_W // NHEAD)

    def stage_slices(s):
        qrow0 = pl.multiple_of(qrow_w + s * _SQROWS, _SQROWS)
        return (i01_hbm.at[pl.ds(qrow0 * 2, _SIROWS)],
                c01_hbm.at[pl.ds(qrow0 * 256, _SELEM)])

    def stage_start(s, sbuf):
        i01_v, c01_v, sem = sbuf
        isl, csl = stage_slices(s)
        pltpu.async_copy(isl, i01_v, sem)
        pltpu.async_copy(csl, c01_v, sem)

    def stage_wait(s, sbuf):
        i01_v, c01_v, sem = sbuf
        isl, csl = stage_slices(s)
        pltpu.make_async_copy(isl, i01_v, sem).wait()
        pltpu.make_async_copy(csl, c01_v, sem).wait()

    def fire(lc, sbuf, rbuf):
        i01_v, _, _ = sbuf
        rows_v, rsem = rbuf
        for j in range(_GROWS):
            pltpu.async_copy(v_hbm.at[i01_v.at[lc * _GROWS + j]],
                             rows_v.at[pl.ds(j * 128, 128)], rsem)

    def drain_blend(c, lc, sbuf, rbuf, obuf, first_store):
        i01_v, c01_v, _ = sbuf
        rows_v, rsem = rbuf
        out_v, osem = obuf
        base_row = pl.multiple_of(base_row_w + c * _CB, _CB)
        for j in range(_GROWS):
            pltpu.make_async_copy(v_hbm.at[i01_v.at[lc * _GROWS + j]],
                                  rows_v.at[pl.ds(j * 128, 128)], rsem).wait()

        @pl.when(jnp.logical_not(first_store))
        def _():
            pltpu.make_async_copy(out_v, out_hbm.at[pl.ds(base_row, _CB)],
                                  osem).wait()

        cbase = lc * _EPC

        def blend(r, _):
            e0 = cbase + (r // NHEAD) * 256 + (r % NHEAD) * 16
            acc_lo = jnp.zeros((16,), jnp.float32)
            acc_hi = jnp.zeros((16,), jnp.float32)
            for k in range(NLVL * NPTS):
                s0 = e0 + k
                s1 = s0 + 128
                c0vec = plsc.load_gather(c01_v, [jnp.full((16,), 0, jnp.int32) + s0])
                c1vec = plsc.load_gather(c01_v, [jnp.full((16,), 0, jnp.int32) + s1])
                r0 = (r // NHEAD) * 256 + (r % NHEAD) * 16 + k
                r1 = r0 + 128
                acc_lo = (acc_lo + c0vec * rows_v[r0, pl.ds(0, 16)]
                          + c1vec * rows_v[r1, pl.ds(0, 16)])
                acc_hi = (acc_hi + c0vec * rows_v[r0, pl.ds(16, 16)]
                          + c1vec * rows_v[r1, pl.ds(16, 16)])
            out_v[r, pl.ds(0, 16)] = acc_lo
            out_v[r, pl.ds(16, 16)] = acc_hi
            return 0

        lax.fori_loop(0, _CB, blend, 0, unroll=2)
        pltpu.async_copy(out_v, out_hbm.at[pl.ds(base_row, _CB)], osem)

    stage_start(0, stages[0])
    stage_start(1, stages[1])
    stage_wait(0, stages[0])
    fire(0, stages[0], rows[0])

    def super_block(sp, sup_par):
        s = 2 * sp + sup_par
        s_cur = stages[sup_par]
        s_next = stages[1 - sup_par]

        def g_step(g2, _):
            for t in (0, 1):
                lc = 2 * g2 + t
                c = s * _CPS + lc
                if t == 0:
                    fire(lc + 1, s_cur, rows[1])
                else:
                    @pl.when(g2 < 3)
                    def _():
                        fire(lc + 1, s_cur, rows[0])

                    @pl.when(jnp.logical_and(g2 == 3, s < _NSUP - 1))
                    def _():
                        stage_wait(s + 1, s_next)
                        fire(0, s_next, rows[0])
                drain_blend(c, lc, s_cur, rows[t], outs[t],
                            first_store=(c == t))
            return 0

        lax.fori_loop(0, _CPS // 2, g_step, 0)
        @pl.when(jnp.asarray(s + 2 < _NSUP))
        def _():
            stage_start(s + 2, s_cur)

    def sp_step(sp, _):
        super_block(sp, 0)
        super_block(sp, 1)
        return 0

    lax.fori_loop(0, _NSUP // 2, sp_step, 0)
    for t in (0, 1):
        out_v, osem = outs[t]
        base_row = pl.multiple_of(base_row_w, _CB)
        pltpu.make_async_copy(out_v, out_hbm.at[pl.ds(base_row, _CB)],
                              osem).wait()


def kernel(query, reference_points, input_flatten, input_spatial_shapes,
           input_level_start_index, Wv, bv, Ws, bs, Wa, ba, Wo, bo):
    x = input_flatten.reshape(N * LEN_IN, DM)
    v = _mm_bias(x, Wv.T, bv, 1024)
    v_tab = v.reshape(NROWS_V, DH)

    q2 = query.reshape(NQ, DM)
    rp2 = reference_points.reshape(NQ, NLVL)
    rp128 = jnp.tile(jnp.repeat(rp2, NPTS, axis=1), (1, NHEAD))
    c01, i01 = _prep(q2, rp128, Ws.T, bs, Wa.T, ba)

    s = _get_sc_sample()(v_tab, i01.reshape(NQ * 2, 128), c01.reshape(-1))

    out = _mm_bias(s.reshape(NQ, DM), Wo.T, bo, 1024)
    return out.reshape(N, LQ, DM)


# speedup vs baseline: 1.8470x; 1.0232x over previous
import functools

import jax
import jax.numpy as jnp
import numpy as np
from jax import lax
from jax.experimental import pallas as pl
from jax.experimental.pallas import tpu as pltpu
from jax.experimental.pallas import tpu_sc as plsc

N = 2
LQ = 4096
DM = 256
NHEAD = 8
DH = DM // NHEAD
NLVL = 4
NPTS = 4
SHAPES = (4096, 2048, 1024, 512)
STARTS = (0, 4096, 6144, 7168)
LEN_IN = 7680

NQ = N * LQ
NQH = NQ * NHEAD
NROWS_V = N * LEN_IN * NHEAD

_NC, _NS = 2, 16
_NW = _NC * _NS
_ROWS_PER_W = NQH // _NW
_CB = 32
_NCHUNK = _ROWS_PER_W // _CB
_SPC = _CB * NLVL * NPTS
_QPC = _CB // NHEAD
_EPC = _QPC * 2 * 128
_GROWS = _EPC // 128

_lane = np.arange(128)
_lvl = (_lane // 4) % 4
_TVEC = np.array(SHAPES, np.float32)[_lvl]
_STARTH = (np.array(STARTS, np.int64)[_lvl] * NHEAD).astype(np.int32)
_HLANE = (_lane // 16).astype(np.int32)
_BD = (_lane[:, None] // 16 == _lane[None, :] // 16).astype(np.float32)


def _mm_bias_body(x_ref, w_ref, b_ref, o_ref):
    o_ref[...] = (
        jnp.dot(x_ref[...], w_ref[...], preferred_element_type=jnp.float32)
        + b_ref[...]
    ).astype(o_ref.dtype)


def _mm_bias(x, w_t, b, bm, out_dtype=jnp.float32):
    m = x.shape[0]
    k = x.shape[1]
    n_out = w_t.shape[1]
    return pl.pallas_call(
        _mm_bias_body,
        grid=(m // bm,),
        in_specs=[
            pl.BlockSpec((bm, k), lambda i: (i, 0)),
            pl.BlockSpec((k, n_out), lambda i: (0, 0)),
            pl.BlockSpec((1, n_out), lambda i: (0, 0)),
        ],
        out_specs=pl.BlockSpec((bm, n_out), lambda i: (i, 0)),
        out_shape=jax.ShapeDtypeStruct((m, n_out), out_dtype),
    )(x, w_t, b.reshape(1, n_out))


_PREP_BM = 1024


def _prep_body(q_ref, rp_ref, wst_ref, bs_ref, wat_ref, ba_ref,
               tv_ref, sh_ref, hl_ref, bd_ref,
               c01_ref, i01_ref):
    pid = pl.program_id(0)
    q = q_ref[...]
    off = jnp.dot(q, wst_ref[...], preferred_element_type=jnp.float32,
                  precision=jax.lax.Precision.HIGHEST) + bs_ref[...]
    logits = jnp.dot(q, wat_ref[...], preferred_element_type=jnp.float32,
                     precision=jax.lax.Precision.HIGHEST) + ba_ref[...]
    m = jnp.max(logits, axis=-1, keepdims=True)
    e = jnp.exp(logits - m)
    denom = jnp.dot(e, bd_ref[...], preferred_element_type=jnp.float32,
                    precision=jax.lax.Precision.HIGHEST)
    aw = e / denom
    refb = rp_ref[...]
    tvec = tv_ref[...]
    loc = refb + off / tvec
    ix = jnp.clip(loc * tvec - 0.5, 0.0, tvec - 1.0)
    i0f = jnp.floor(ix)
    w = ix - i0f
    i0 = i0f.astype(jnp.int32)
    i1 = jnp.minimum(i0 + 1, tvec.astype(jnp.int32) - 1)
    nbase = (pid // (LQ // _PREP_BM)) * (LEN_IN * NHEAD)
    idx0 = nbase + sh_ref[...] + i0 * NHEAD + hl_ref[...]
    idx1 = nbase + sh_ref[...] + i1 * NHEAD + hl_ref[...]
    c01_ref[...] = jnp.concatenate([aw * (1.0 - w), aw * w], axis=1)
    i01_ref[...] = jnp.concatenate([idx0, idx1], axis=1)


def _prep(q2, rp128, ws_t, bs, wa_t, ba):
    vec_spec = pl.BlockSpec((1, 128), lambda i: (0, 0))
    blk128 = pl.BlockSpec((_PREP_BM, 128), lambda i: (i, 0))
    blk256 = pl.BlockSpec((_PREP_BM, 256), lambda i: (i, 0))
    return pl.pallas_call(
        _prep_body,
        grid=(NQ // _PREP_BM,),
        in_specs=[
            pl.BlockSpec((_PREP_BM, DM), lambda i: (i, 0)),
            blk128,
            pl.BlockSpec((DM, 128), lambda i: (0, 0)),
            vec_spec,
            pl.BlockSpec((DM, 128), lambda i: (0, 0)),
            vec_spec,
            vec_spec, vec_spec, vec_spec,
            pl.BlockSpec((128, 128), lambda i: (0, 0)),
        ],
        out_specs=[blk256, blk256],
        out_shape=[
            jax.ShapeDtypeStruct((NQ, 256), jnp.float32),
            jax.ShapeDtypeStruct((NQ, 256), jnp.int32),
        ],
    )(q2, rp128, ws_t, bs.reshape(1, 128), wa_t, ba.reshape(1, 128),
      _TVEC.reshape(1, 128), _STARTH.reshape(1, 128), _HLANE.reshape(1, 128),
      _BD)


_CPS = 8
_NSUP = _NCHUNK // _CPS
_SQROWS = _QPC * _CPS
_SIROWS = _SQROWS * 2
_SELEM = _SQROWS * 256


@functools.cache
def _get_sc_sample():
    mesh = plsc.VectorSubcoreMesh(core_axis_name="c", subcore_axis_name="s")
    stage = lambda: (
        pltpu.VMEM((_SIROWS, 128), jnp.int32),
        pltpu.VMEM((_SELEM,), jnp.float32),
        pltpu.SemaphoreType.DMA,
    )

    @functools.partial(
        pl.kernel,
        mesh=mesh,
        compiler_params=pltpu.CompilerParams(
            needs_layout_passes=False, use_tc_tiling_on_sc=False),
        out_type=jax.ShapeDtypeStruct((NQH, DH), jnp.float32),
        scratch_types=[
            *stage(), *stage(),
            pltpu.VMEM((_EPC, DH), jnp.float32), pltpu.SemaphoreType.DMA,
            pltpu.VMEM((_EPC, DH), jnp.float32), pltpu.SemaphoreType.DMA,
            pltpu.VMEM((_CB, DH), jnp.float32), pltpu.SemaphoreType.DMA,
            pltpu.VMEM((_CB, DH), jnp.float32), pltpu.SemaphoreType.DMA,
        ],
    )
    def _sc_sample(v_hbm, i01_hbm, c01_hbm, out_hbm,
                   iA, cA, sA, iB, cB, sB,
                   r0, rs0, r1, rs1, o0, os0, o1, os1):
        _sc_body(v_hbm, i01_hbm, c01_hbm, out_hbm,
                 ((iA, cA, sA), (iB, cB, sB)),
                 ((r0, rs0), (r1, rs1)), ((o0, os0), (o1, os1)))

    return _sc_sample


def _sc_body(v_hbm, i01_hbm, c01_hbm, out_hbm, stages, rows, outs):
    wid = lax.axis_index("s") * _NC + lax.axis_index("c")
    base_row_w = wid * _ROWS_PER_W
    qrow_w = wid * (_ROWS_PER_W // NHEAD)

    def stage_slices(s):
        qrow0 = pl.multiple_of(qrow_w + s * _SQROWS, _SQROWS)
        return (i01_hbm.at[pl.ds(qrow0 * 2, _SIROWS)],
                c01_hbm.at[pl.ds(qrow0 * 256, _SELEM)])

    def stage_start(s, sbuf):
        i01_v, c01_v, sem = sbuf
        isl, csl = stage_slices(s)
        pltpu.async_copy(isl, i01_v, sem)
        pltpu.async_copy(csl, c01_v, sem)

    def stage_wait(s, sbuf):
        i01_v, c01_v, sem = sbuf
        isl, csl = stage_slices(s)
        pltpu.make_async_copy(isl, i01_v, sem).wait()
        pltpu.make_async_copy(csl, c01_v, sem).wait()

    def fire(lc, sbuf, rbuf):
        i01_v, _, _ = sbuf
        rows_v, rsem = rbuf
        for j in range(_GROWS):
            pltpu.async_copy(v_hbm.at[i01_v.at[lc * _GROWS + j]],
                             rows_v.at[pl.ds(j * 128, 128)], rsem)

    def drain_blend(c, lc, sbuf, rbuf, obuf, first_store):
        i01_v, c01_v, _ = sbuf
        rows_v, rsem = rbuf
        out_v, osem = obuf
        base_row = pl.multiple_of(base_row_w + c * _CB, _CB)
        for j in range(_GROWS):
            pltpu.make_async_copy(v_hbm.at[i01_v.at[lc * _GROWS + j]],
                                  rows_v.at[pl.ds(j * 128, 128)], rsem).wait()

        @pl.when(jnp.logical_not(first_store))
        def _():
            pltpu.make_async_copy(out_v, out_hbm.at[pl.ds(base_row, _CB)],
                                  osem).wait()

        cbase = lc * _EPC

        def blend(r, _):
            e0 = cbase + (r // NHEAD) * 256 + (r % NHEAD) * 16
            acc_lo = jnp.zeros((16,), jnp.float32)
            acc_hi = jnp.zeros((16,), jnp.float32)
            for k in range(NLVL * NPTS):
                s0 = e0 + k
                s1 = s0 + 128
                c0vec = plsc.load_gather(c01_v, [jnp.full((16,), 0, jnp.int32) + s0])
                c1vec = plsc.load_gather(c01_v, [jnp.full((16,), 0, jnp.int32) + s1])
                r0 = (r // NHEAD) * 256 + (r % NHEAD) * 16 + k
                r1 = r0 + 128
                acc_lo = (acc_lo + c0vec * rows_v[r0, pl.ds(0, 16)]
                          + c1vec * rows_v[r1, pl.ds(0, 16)])
                acc_hi = (acc_hi + c0vec * rows_v[r0, pl.ds(16, 16)]
                          + c1vec * rows_v[r1, pl.ds(16, 16)])
            out_v[r, pl.ds(0, 16)] = acc_lo
            out_v[r, pl.ds(16, 16)] = acc_hi
            return 0

        lax.fori_loop(0, _CB, blend, 0, unroll=2)
        pltpu.async_copy(out_v, out_hbm.at[pl.ds(base_row, _CB)], osem)

    stage_start(0, stages[0])
    stage_start(1, stages[1])
    stage_wait(0, stages[0])
    fire(0, stages[0], rows[0])

    def super_block(sp, sup_par):
        s = 2 * sp + sup_par
        s_cur = stages[sup_par]
        s_next = stages[1 - sup_par]

        def g_step(g2, _):
            for t in (0, 1):
                lc = 2 * g2 + t
                c = s * _CPS + lc
                if t == 0:
                    fire(lc + 1, s_cur, rows[1])
                else:
                    @pl.when(g2 < 3)
                    def _():
                        fire(lc + 1, s_cur, rows[0])

                    @pl.when(jnp.logical_and(g2 == 3, s < _NSUP - 1))
                    def _():
                        stage_wait(s + 1, s_next)
                        fire(0, s_next, rows[0])
                drain_blend(c, lc, s_cur, rows[t], outs[t],
                            first_store=(c == t))
            return 0

        lax.fori_loop(0, _CPS // 2, g_step, 0)
        @pl.when(jnp.asarray(s + 2 < _NSUP))
        def _():
            stage_start(s + 2, s_cur)

    def sp_step(sp, _):
        super_block(sp, 0)
        super_block(sp, 1)
        return 0

    lax.fori_loop(0, _NSUP // 2, sp_step, 0)
    for t in (0, 1):
        out_v, osem = outs[t]
        base_row = pl.multiple_of(base_row_w, _CB)
        pltpu.make_async_copy(out_v, out_hbm.at[pl.ds(base_row, _CB)],
                              osem).wait()


def kernel(query, reference_points, input_flatten, input_spatial_shapes,
           input_level_start_index, Wv, bv, Ws, bs, Wa, ba, Wo, bo):
    x = input_flatten.reshape(N * LEN_IN, DM)
    v = _mm_bias(x, Wv.T, bv, 1024)
    v_tab = v.reshape(NROWS_V, DH)

    q2 = query.reshape(NQ, DM)
    rp2 = reference_points.reshape(NQ, NLVL)
    rp128 = jnp.tile(jnp.repeat(rp2, NPTS, axis=1), (1, NHEAD))
    c01, i01 = _prep(q2, rp128, Ws.T, bs, Wa.T, ba)

    s = _get_sc_sample()(v_tab, i01.reshape(NQ * 2, 128), c01.reshape(-1))

    out = _mm_bias(s.reshape(NQ, DM), Wo.T, bo, 1024)
    return out.reshape(N, LQ, DM)


# speedup vs baseline: 1.9269x; 1.0432x over previous
import functools

import jax
import jax.numpy as jnp
import numpy as np
from jax import lax
from jax.experimental import pallas as pl
from jax.experimental.pallas import tpu as pltpu
from jax.experimental.pallas import tpu_sc as plsc

N = 2
LQ = 4096
DM = 256
NHEAD = 8
DH = DM // NHEAD
NLVL = 4
NPTS = 4
SHAPES = (4096, 2048, 1024, 512)
STARTS = (0, 4096, 6144, 7168)
LEN_IN = 7680

NQ = N * LQ
NQH = NQ * NHEAD
NROWS_V = N * LEN_IN * NHEAD

_NC, _NS = 2, 16
_NW = _NC * _NS
_ROWS_PER_W = NQH // _NW
_CB = 32
_NCHUNK = _ROWS_PER_W // _CB
_SPC = _CB * NLVL * NPTS
_QPC = _CB // NHEAD
_EPC = _QPC * 2 * 128
_GROWS = _EPC // 128

_lane = np.arange(128)
_lvl = (_lane // 4) % 4
_TVEC = np.array(SHAPES, np.float32)[_lvl]
_STARTH = (np.array(STARTS, np.int64)[_lvl] * NHEAD).astype(np.int32)
_HLANE = (_lane // 16).astype(np.int32)
_BD = (_lane[:, None] // 16 == _lane[None, :] // 16).astype(np.float32)


def _mm_bias_body(x_ref, w_ref, b_ref, o_ref):
    o_ref[...] = (
        jnp.dot(x_ref[...], w_ref[...], preferred_element_type=jnp.float32)
        + b_ref[...]
    ).astype(o_ref.dtype)


def _mm_bias(x, w_t, b, bm, out_dtype=jnp.float32):
    m = x.shape[0]
    k = x.shape[1]
    n_out = w_t.shape[1]
    return pl.pallas_call(
        _mm_bias_body,
        grid=(m // bm,),
        in_specs=[
            pl.BlockSpec((bm, k), lambda i: (i, 0)),
            pl.BlockSpec((k, n_out), lambda i: (0, 0)),
            pl.BlockSpec((1, n_out), lambda i: (0, 0)),
        ],
        out_specs=pl.BlockSpec((bm, n_out), lambda i: (i, 0)),
        out_shape=jax.ShapeDtypeStruct((m, n_out), out_dtype),
    )(x, w_t, b.reshape(1, n_out))


_PREP_BM = 1024


def _prep_body(q_ref, rp_ref, wst_ref, bs_ref, wat_ref, ba_ref,
               tv_ref, sh_ref, hl_ref, bd_ref,
               c01_ref, i01_ref):
    pid = pl.program_id(0)
    q = q_ref[...]
    off = jnp.dot(q, wst_ref[...], preferred_element_type=jnp.float32) + bs_ref[...]
    logits = jnp.dot(q, wat_ref[...], preferred_element_type=jnp.float32) + ba_ref[...]
    m = jnp.max(logits, axis=-1, keepdims=True)
    e = jnp.exp(logits - m)
    denom = jnp.dot(e, bd_ref[...], preferred_element_type=jnp.float32,
                    precision=jax.lax.Precision.HIGHEST)
    aw = e / denom
    refb = rp_ref[...]
    tvec = tv_ref[...]
    loc = refb + off / tvec
    ix = jnp.clip(loc * tvec - 0.5, 0.0, tvec - 1.0)
    i0f = jnp.floor(ix)
    w = ix - i0f
    i0 = i0f.astype(jnp.int32)
    i1 = jnp.minimum(i0 + 1, tvec.astype(jnp.int32) - 1)
    nbase = (pid // (LQ // _PREP_BM)) * (LEN_IN * NHEAD)
    idx0 = nbase + sh_ref[...] + i0 * NHEAD + hl_ref[...]
    idx1 = nbase + sh_ref[...] + i1 * NHEAD + hl_ref[...]
    c01_ref[...] = jnp.concatenate([aw * (1.0 - w), aw * w], axis=1)
    i01_ref[...] = jnp.concatenate([idx0, idx1], axis=1)


def _prep(q2, rp128, ws_t, bs, wa_t, ba):
    vec_spec = pl.BlockSpec((1, 128), lambda i: (0, 0))
    blk128 = pl.BlockSpec((_PREP_BM, 128), lambda i: (i, 0))
    blk256 = pl.BlockSpec((_PREP_BM, 256), lambda i: (i, 0))
    return pl.pallas_call(
        _prep_body,
        grid=(NQ // _PREP_BM,),
        in_specs=[
            pl.BlockSpec((_PREP_BM, DM), lambda i: (i, 0)),
            blk128,
            pl.BlockSpec((DM, 128), lambda i: (0, 0)),
            vec_spec,
            pl.BlockSpec((DM, 128), lambda i: (0, 0)),
            vec_spec,
            vec_spec, vec_spec, vec_spec,
            pl.BlockSpec((128, 128), lambda i: (0, 0)),
        ],
        out_specs=[blk256, blk256],
        out_shape=[
            jax.ShapeDtypeStruct((NQ, 256), jnp.float32),
            jax.ShapeDtypeStruct((NQ, 256), jnp.int32),
        ],
    )(q2, rp128, ws_t, bs.reshape(1, 128), wa_t, ba.reshape(1, 128),
      _TVEC.reshape(1, 128), _STARTH.reshape(1, 128), _HLANE.reshape(1, 128),
      _BD)


_CPS = 8
_NSUP = _NCHUNK // _CPS
_SQROWS = _QPC * _CPS
_SIROWS = _SQROWS * 2
_SELEM = _SQROWS * 256


@functools.cache
def _get_sc_sample():
    mesh = plsc.VectorSubcoreMesh(core_axis_name="c", subcore_axis_name="s")
    stage = lambda: (
        pltpu.VMEM((_SIROWS, 128), jnp.int32),
        pltpu.VMEM((_SELEM,), jnp.float32),
        pltpu.SemaphoreType.DMA,
    )

    @functools.partial(
        pl.kernel,
        mesh=mesh,
        compiler_params=pltpu.CompilerParams(
            needs_layout_passes=False, use_tc_tiling_on_sc=False),
        out_type=jax.ShapeDtypeStruct((NQH, DH), jnp.float32),
        scratch_types=[
            *stage(), *stage(),
            pltpu.VMEM((_EPC, DH), jnp.float32), pltpu.SemaphoreType.DMA,
            pltpu.VMEM((_EPC, DH), jnp.float32), pltpu.SemaphoreType.DMA,
            pltpu.VMEM((_CB, DH), jnp.float32), pltpu.SemaphoreType.DMA,
            pltpu.VMEM((_CB, DH), jnp.float32), pltpu.SemaphoreType.DMA,
        ],
    )
    def _sc_sample(v_hbm, i01_hbm, c01_hbm, out_hbm,
                   iA, cA, sA, iB, cB, sB,
                   r0, rs0, r1, rs1, o0, os0, o1, os1):
        _sc_body(v_hbm, i01_hbm, c01_hbm, out_hbm,
                 ((iA, cA, sA), (iB, cB, sB)),
                 ((r0, rs0), (r1, rs1)), ((o0, os0), (o1, os1)))

    return _sc_sample


def _sc_body(v_hbm, i01_hbm, c01_hbm, out_hbm, stages, rows, outs):
    wid = lax.axis_index("s") * _NC + lax.axis_index("c")
    base_row_w = wid * _ROWS_PER_W
    qrow_w = wid * (_ROWS_PER_W // NHEAD)

    def stage_slices(s):
        qrow0 = pl.multiple_of(qrow_w + s * _SQROWS, _SQROWS)
        return (i01_hbm.at[pl.ds(qrow0 * 2, _SIROWS)],
                c01_hbm.at[pl.ds(qrow0 * 256, _SELEM)])

    def stage_start(s, sbuf):
        i01_v, c01_v, sem = sbuf
        isl, csl = stage_slices(s)
        pltpu.async_copy(isl, i01_v, sem)
        pltpu.async_copy(csl, c01_v, sem)

    def stage_wait(s, sbuf):
        i01_v, c01_v, sem = sbuf
        isl, csl = stage_slices(s)
        pltpu.make_async_copy(isl, i01_v, sem).wait()
        pltpu.make_async_copy(csl, c01_v, sem).wait()

    def fire(lc, sbuf, rbuf):
        i01_v, _, _ = sbuf
        rows_v, rsem = rbuf
        for j in range(_GROWS):
            pltpu.async_copy(v_hbm.at[i01_v.at[lc * _GROWS + j]],
                             rows_v.at[pl.ds(j * 128, 128)], rsem)

    def drain_blend(c, lc, sbuf, rbuf, obuf, first_store):
        i01_v, c01_v, _ = sbuf
        rows_v, rsem = rbuf
        out_v, osem = obuf
        base_row = pl.multiple_of(base_row_w + c * _CB, _CB)
        for j in range(_GROWS):
            pltpu.make_async_copy(v_hbm.at[i01_v.at[lc * _GROWS + j]],
                                  rows_v.at[pl.ds(j * 128, 128)], rsem).wait()

        @pl.when(jnp.logical_not(first_store))
        def _():
            pltpu.make_async_copy(out_v, out_hbm.at[pl.ds(base_row, _CB)],
                                  osem).wait()

        cbase = lc * _EPC

        def blend(r, _):
            e0 = cbase + (r // NHEAD) * 256 + (r % NHEAD) * 16
            acc_lo = jnp.zeros((16,), jnp.float32)
            acc_hi = jnp.zeros((16,), jnp.float32)
            for k in range(NLVL * NPTS):
                s0 = e0 + k
                s1 = s0 + 128
                c0vec = plsc.load_gather(c01_v, [jnp.full((16,), 0, jnp.int32) + s0])
                c1vec = plsc.load_gather(c01_v, [jnp.full((16,), 0, jnp.int32) + s1])
                r0 = (r // NHEAD) * 256 + (r % NHEAD) * 16 + k
                r1 = r0 + 128
                acc_lo = (acc_lo + c0vec * rows_v[r0, pl.ds(0, 16)]
                          + c1vec * rows_v[r1, pl.ds(0, 16)])
                acc_hi = (acc_hi + c0vec * rows_v[r0, pl.ds(16, 16)]
                          + c1vec * rows_v[r1, pl.ds(16, 16)])
            out_v[r, pl.ds(0, 16)] = acc_lo
            out_v[r, pl.ds(16, 16)] = acc_hi
            return 0

        lax.fori_loop(0, _CB, blend, 0, unroll=2)
        pltpu.async_copy(out_v, out_hbm.at[pl.ds(base_row, _CB)], osem)

    stage_start(0, stages[0])
    stage_start(1, stages[1])
    stage_wait(0, stages[0])
    fire(0, stages[0], rows[0])

    def super_block(sp, sup_par):
        s = 2 * sp + sup_par
        s_cur = stages[sup_par]
        s_next = stages[1 - sup_par]

        def g_step(g2, _):
            for t in (0, 1):
                lc = 2 * g2 + t
                c = s * _CPS + lc
                if t == 0:
                    fire(lc + 1, s_cur, rows[1])
                else:
                    @pl.when(g2 < 3)
                    def _():
                        fire(lc + 1, s_cur, rows[0])

                    @pl.when(jnp.logical_and(g2 == 3, s < _NSUP - 1))
                    def _():
                        stage_wait(s + 1, s_next)
                        fire(0, s_next, rows[0])
                drain_blend(c, lc, s_cur, rows[t], outs[t],
                            first_store=(c == t))
            return 0

        lax.fori_loop(0, _CPS // 2, g_step, 0)
        @pl.when(jnp.asarray(s + 2 < _NSUP))
        def _():
            stage_start(s + 2, s_cur)

    def sp_step(sp, _):
        super_block(sp, 0)
        super_block(sp, 1)
        return 0

    lax.fori_loop(0, _NSUP // 2, sp_step, 0)
    for t in (0, 1):
        out_v, osem = outs[t]
        base_row = pl.multiple_of(base_row_w, _CB)
        pltpu.make_async_copy(out_v, out_hbm.at[pl.ds(base_row, _CB)],
                              osem).wait()


def kernel(query, reference_points, input_flatten, input_spatial_shapes,
           input_level_start_index, Wv, bv, Ws, bs, Wa, ba, Wo, bo):
    x = input_flatten.reshape(N * LEN_IN, DM)
    v = _mm_bias(x, Wv.T, bv, 1024)
    v_tab = v.reshape(NROWS_V, DH)

    q2 = query.reshape(NQ, DM)
    rp2 = reference_points.reshape(NQ, NLVL)
    rp128 = jnp.tile(jnp.repeat(rp2, NPTS, axis=1), (1, NHEAD))
    c01, i01 = _prep(q2, rp128, Ws.T, bs, Wa.T, ba)

    s = _get_sc_sample()(v_tab, i01.reshape(NQ * 2, 128), c01.reshape(-1))

    out = _mm_bias(s.reshape(NQ, DM), Wo.T, bo, 1024)
    return out.reshape(N, LQ, DM)
